# baseline (speedup 1.0000x reference)
import functools
import jax
import jax.numpy as jnp
from jax import lax
from jax.experimental import pallas as pl
from jax.experimental.pallas import tpu as pltpu
from jax.experimental.pallas import tpu_sc as plsc

N = 10000
NP = 10240
E = 160000
EP = 163840
EROWS = EP // 128
D0, D1, D2 = 256, 256, 128
NC, NS = 2, 16
RT = NP // NS
K = 128
CH1 = EP // NS // K
CH2 = EP // (NC * NS) // K

_MESH = plsc.VectorSubcoreMesh(core_axis_name="c", subcore_axis_name="s",
                               num_cores=NC, num_subcores=NS)


def _deg_body(dst2d, zeros_k, ones_k, p0, p1, stage_d, ones_v, acc):
  cid = lax.axis_index("c")
  sid = lax.axis_index("s")
  r0 = sid * RT

  pltpu.sync_copy(zeros_k, ones_v)

  def init_chunk(j, carry):
    pltpu.sync_copy(ones_v, acc.at[pl.ds(r0 + j * K, K)])
    return carry

  lax.fori_loop(0, RT // K, init_chunk, 0)
  pltpu.sync_copy(ones_k, ones_v)
  pltpu.sync_copy(dst2d.at[pl.ds((cid * NS + sid) * CH2, CH2)], stage_d)
  plsc.subcore_barrier()

  def chunk(j, carry):
    pltpu.sync_copy(ones_v, acc.at[stage_d.at[j]], add=True)
    return carry

  lax.fori_loop(0, CH2, chunk, 0)
  plsc.subcore_barrier()

  def out_chunk(j, carry):
    sl = pl.ds(r0 + j * K, K)
    pltpu.sync_copy(acc.at[sl], ones_v)

    @pl.when(cid == 0)
    def _():
      pltpu.sync_copy(ones_v, p0.at[sl])

    @pl.when(cid == 1)
    def _():
      pltpu.sync_copy(ones_v, p1.at[sl])

    return carry

  lax.fori_loop(0, RT // K, out_chunk, 0)


def _deg_counts(dst2d):
  zeros_k = jnp.zeros((K, 128), jnp.float32)
  ones_k = jnp.ones((K, 128), jnp.float32)
  f = pl.kernel(
      _deg_body,
      out_type=(jax.ShapeDtypeStruct((NP, 128), jnp.float32),
                jax.ShapeDtypeStruct((NP, 128), jnp.float32)),
      mesh=_MESH,
      scratch_types=[
          pltpu.VMEM((CH2, 128), jnp.int32),
          pltpu.VMEM((K, 128), jnp.float32),
          pltpu.VMEM_SHARED((NP, 128), jnp.float32),
      ],
  )
  return f(dst2d, zeros_k, ones_k)


def _msg_edge_loop(gather_from, srcp, stage_d, e0, svs, bufs, gsems, isems,
                   acc, ch):
  def idx_issue(c, q):
    pltpu.async_copy(srcp.at[pl.ds(e0 + c * K, K)], svs[q], isems[q])

  def idx_wait(q):
    pltpu.make_async_copy(srcp.at[pl.ds(e0, K)], svs[q], isems[q]).wait()

  for q in range(4):
    idx_issue(q, q)
  for b in range(2):
    idx_wait(b)
    pltpu.async_copy(gather_from.at[svs[b]], bufs[b], gsems[b])

  def chunk(i, carry):
    j = i * 4
    for u in range(4):
      jb = j + u
      b = u % 2
      q = u
      qn = (u + 2) % 4
      pltpu.make_async_copy(gather_from.at[svs[q]], bufs[b],
                            gsems[b]).wait()
      pltpu.sync_copy(bufs[b], acc.at[stage_d.at[jb]], add=True)

      @pl.when(jb + 4 < ch)
      def _():
        idx_issue(jb + 4, q)

      @pl.when(jb + 2 < ch)
      def _():
        idx_wait(qn)
        pltpu.async_copy(gather_from.at[svs[qn]], bufs[b], gsems[b])

    return carry

  lax.fori_loop(0, ch // 4, chunk, 0)


def _init_acc(src_of, hbm_ref, acc, r0, bufs, gsems):
  nchunk = RT // K
  src_of(0, bufs[0], gsems[0])
  for r in range(nchunk):
    b = r % 2
    pltpu.make_async_copy(hbm_ref.at[pl.ds(r0, K)], bufs[b],
                          gsems[b]).wait()
    if r + 1 < nchunk:
      src_of(r + 1, bufs[(r + 1) % 2], gsems[(r + 1) % 2])
    pltpu.sync_copy(bufs[b], acc.at[pl.ds(r0 + r * K, K)])


def _writeout(acc, r0, dst_of, out_ref, bufs, gsems):
  nchunk = RT // K
  for r in range(nchunk):
    b = r % 2
    if r >= 2:
      pltpu.make_async_copy(bufs[b], out_ref.at[pl.ds(r0, K)],
                            gsems[b]).wait()
    pltpu.sync_copy(acc.at[pl.ds(r0 + r * K, K)], bufs[b])
    dst_of(r, bufs[b], gsems[b])
  for r in range(max(0, nchunk - 2), nchunk):
    b = r % 2
    pltpu.make_async_copy(bufs[b], out_ref.at[pl.ds(r0, K)],
                          gsems[b]).wait()


def _msg1_body(ya, yb, srcp, dst2d, oa, ob, sv0, sv1, sv2, sv3, stage_d,
               buf0, buf1, acc, gsem0, gsem1, isem0, isem1, isem2, isem3):
  cid = lax.axis_index("c")
  sid = lax.axis_index("s")
  r0 = sid * RT
  bufs = (buf0, buf1)
  gsems = (gsem0, gsem1)

  def src_of(r, buf, sem):
    sl = pl.ds(r0 + r * K, K)

    @pl.when(cid == 0)
    def _():
      pltpu.async_copy(ya.at[sl], buf, sem)

    @pl.when(cid == 1)
    def _():
      pltpu.async_copy(yb.at[sl], buf, sem)

  _init_acc(src_of, ya, acc, r0, bufs, gsems)
  pltpu.sync_copy(dst2d.at[pl.ds(sid * CH1, CH1)], stage_d)
  plsc.subcore_barrier()

  e0 = sid * (EP // NS)

  @pl.when(cid == 0)
  def _():
    _msg_edge_loop(ya, srcp, stage_d, e0, (sv0, sv1, sv2, sv3), bufs, gsems,
                   (isem0, isem1, isem2, isem3), acc, CH1)

  @pl.when(cid == 1)
  def _():
    _msg_edge_loop(yb, srcp, stage_d, e0, (sv0, sv1, sv2, sv3), bufs, gsems,
                   (isem0, isem1, isem2, isem3), acc, CH1)

  plsc.subcore_barrier()

  def dst_of(r, buf, sem):
    sl = pl.ds(r0 + r * K, K)

    @pl.when(cid == 0)
    def _():
      pltpu.async_copy(buf, oa.at[sl], sem)

    @pl.when(cid == 1)
    def _():
      pltpu.async_copy(buf, ob.at[sl], sem)

  _writeout(acc, r0, dst_of, oa, bufs, gsems)


def _message_pass1(ya, yb, srcp, dst2d):
  f = pl.kernel(
      _msg1_body,
      out_type=(jax.ShapeDtypeStruct((NP, 128), jnp.float32),
                jax.ShapeDtypeStruct((NP, 128), jnp.float32)),
      mesh=_MESH,
      scratch_types=[
          pltpu.VMEM((K,), jnp.int32),
          pltpu.VMEM((K,), jnp.int32),
          pltpu.VMEM((K,), jnp.int32),
          pltpu.VMEM((K,), jnp.int32),
          pltpu.VMEM((CH1, 128), jnp.int32),
          pltpu.VMEM((K, 128), jnp.float32),
          pltpu.VMEM((K, 128), jnp.float32),
          pltpu.VMEM_SHARED((NP, 128), jnp.float32),
          pltpu.SemaphoreType.DMA,
          pltpu.SemaphoreType.DMA,
          pltpu.SemaphoreType.DMA,
          pltpu.SemaphoreType.DMA,
          pltpu.SemaphoreType.DMA,
          pltpu.SemaphoreType.DMA,
      ],
  )
  return f(ya, yb, srcp, dst2d)


def _msg2_body(y2, srcp, dst2d, zeros_k, p0, p1, sv0, sv1, sv2, sv3, stage_d,
               buf0, buf1, acc, gsem0, gsem1, isem0, isem1, isem2, isem3):
  cid = lax.axis_index("c")
  sid = lax.axis_index("s")
  r0 = sid * RT
  bufs = (buf0, buf1)
  gsems = (gsem0, gsem1)

  def src_of(r, buf, sem):
    pltpu.async_copy(zeros_k, buf, sem)

  _init_acc(src_of, y2, acc, r0, bufs, gsems)
  pltpu.sync_copy(dst2d.at[pl.ds((cid * NS + sid) * CH2, CH2)], stage_d)
  plsc.subcore_barrier()

  e0 = (cid * NS + sid) * (EP // (NC * NS))
  _msg_edge_loop(y2, srcp, stage_d, e0, (sv0, sv1, sv2, sv3), bufs, gsems,
                 (isem0, isem1, isem2, isem3), acc, CH2)
  plsc.subcore_barrier()

  def dst_of(r, buf, sem):
    sl = pl.ds(r0 + r * K, K)

    @pl.when(cid == 0)
    def _():
      pltpu.async_copy(buf, p0.at[sl], sem)

    @pl.when(cid == 1)
    def _():
      pltpu.async_copy(buf, p1.at[sl], sem)

  _writeout(acc, r0, dst_of, p0, bufs, gsems)


def _message_pass2(y2, srcp, dst2d):
  zeros_k = jnp.zeros((K, 128), jnp.float32)
  f = pl.kernel(
      _msg2_body,
      out_type=(jax.ShapeDtypeStruct((NP, 128), jnp.float32),
                jax.ShapeDtypeStruct((NP, 128), jnp.float32)),
      mesh=_MESH,
      scratch_types=[
          pltpu.VMEM((K,), jnp.int32),
          pltpu.VMEM((K,), jnp.int32),
          pltpu.VMEM((K,), jnp.int32),
          pltpu.VMEM((K,), jnp.int32),
          pltpu.VMEM((CH2, 128), jnp.int32),
          pltpu.VMEM((K, 128), jnp.float32),
          pltpu.VMEM((K, 128), jnp.float32),
          pltpu.VMEM_SHARED((NP, 128), jnp.float32),
          pltpu.SemaphoreType.DMA,
          pltpu.SemaphoreType.DMA,
          pltpu.SemaphoreType.DMA,
          pltpu.SemaphoreType.DMA,
          pltpu.SemaphoreType.DMA,
          pltpu.SemaphoreType.DMA,
      ],
  )
  return f(y2, srcp, dst2d, zeros_k)


_RB = 1000


def _mm1_body(d0, d1, x_ref, w_ref, ya, yb):
  dinv = lax.rsqrt(d0[...] + d1[...] + 1.0)
  y = jnp.dot(x_ref[...], w_ref[...],
              preferred_element_type=jnp.float32) * dinv
  ya[...] = y[:, :D1 // 2]
  yb[...] = y[:, D1 // 2:]


def _layer1_matmul(deg0, deg1, x, W1):
  grid = (N // _RB,)
  return pl.pallas_call(
      _mm1_body,
      grid=grid,
      in_specs=[
          pl.BlockSpec((_RB, 1), lambda i: (i, 0)),
          pl.BlockSpec((_RB, 1), lambda i: (i, 0)),
          pl.BlockSpec((_RB, D0), lambda i: (i, 0)),
          pl.BlockSpec((D0, D1), lambda i: (0, 0)),
      ],
      out_specs=(
          pl.BlockSpec((_RB, D1 // 2), lambda i: (i, 0)),
          pl.BlockSpec((_RB, D1 // 2), lambda i: (i, 0)),
      ),
      out_shape=(
          jax.ShapeDtypeStruct((NP, D1 // 2), jnp.float32),
          jax.ShapeDtypeStruct((NP, D1 // 2), jnp.float32),
      ),
  )(deg0, deg1, x, W1)


def _mm2_body(aa, ab, d0, d1, b1, w_ref, y2):
  dinv = lax.rsqrt(d0[...] + d1[...] + 1.0)
  h = jnp.concatenate([aa[...], ab[...]], axis=1)
  h = jnp.maximum(h * dinv + b1[...], 0.0)
  y2[...] = jnp.dot(h, w_ref[...], preferred_element_type=jnp.float32) * dinv


def _layer2_matmul(acc1a, acc1b, deg0, deg1, b1, W2):
  grid = (N // _RB,)
  return pl.pallas_call(
      _mm2_body,
      grid=grid,
      in_specs=[
          pl.BlockSpec((_RB, D1 // 2), lambda i: (i, 0)),
          pl.BlockSpec((_RB, D1 // 2), lambda i: (i, 0)),
          pl.BlockSpec((_RB, 1), lambda i: (i, 0)),
          pl.BlockSpec((_RB, 1), lambda i: (i, 0)),
          pl.BlockSpec((1, D1), lambda i: (0, 0)),
          pl.BlockSpec((D1, D2), lambda i: (0, 0)),
      ],
      out_specs=pl.BlockSpec((_RB, D2), lambda i: (i, 0)),
      out_shape=jax.ShapeDtypeStruct((NP, D2), jnp.float32),
  )(acc1a, acc1b, deg0, deg1, b1, W2)


def _epi_body(p0, p1, y2, d0, d1, b2, z_ref):
  dinv = lax.rsqrt(d0[...] + d1[...] + 1.0)
  z_ref[...] = (p0[...] + p1[...] + y2[...]) * dinv + b2[...]


def _epilogue(p0, p1, y2, deg0, deg1, b2):
  grid = (N // _RB,)
  return pl.pallas_call(
      _epi_body,
      grid=grid,
      in_specs=[
          pl.BlockSpec((_RB, D2), lambda i: (i, 0)),
          pl.BlockSpec((_RB, D2), lambda i: (i, 0)),
          pl.BlockSpec((_RB, D2), lambda i: (i, 0)),
          pl.BlockSpec((_RB, 1), lambda i: (i, 0)),
          pl.BlockSpec((_RB, 1), lambda i: (i, 0)),
          pl.BlockSpec((1, D2), lambda i: (0, 0)),
      ],
      out_specs=pl.BlockSpec((_RB, D2), lambda i: (i, 0)),
      out_shape=jax.ShapeDtypeStruct((N, D2), jnp.float32),
  )(p0, p1, y2, deg0, deg1, b2)


@jax.jit
def kernel(x, edge_index, W1, b1, W2, b2):
  pad = jnp.full((EP - E,), N, jnp.int32)
  srcp = jnp.concatenate([edge_index[0].astype(jnp.int32), pad])
  dstp = jnp.concatenate([edge_index[1].astype(jnp.int32), pad])
  dst2d = dstp.reshape(EROWS, 128)

  h0, h1 = _deg_counts(dst2d)
  deg0 = h0[:, :1]
  deg1 = h1[:, :1]
  y1a, y1b = _layer1_matmul(deg0, deg1, x, W1)
  acc1a, acc1b = _message_pass1(y1a, y1b, srcp, dst2d)
  y2 = _layer2_matmul(acc1a, acc1b, deg0, deg1, b1.reshape(1, D1), W2)
  p0, p1 = _message_pass2(y2, srcp, dst2d)
  return _epilogue(p0, p1, y2, deg0, deg1, b2.reshape(1, D2))


# speedup vs baseline: 1.0851x; 1.0851x over previous
import functools
import jax
import jax.numpy as jnp
import numpy as np
from jax import lax
from jax.experimental import pallas as pl
from jax.experimental.pallas import tpu as pltpu
from jax.experimental.pallas import tpu_sc as plsc

N = 10000
NP = 10240
E = 160000
EP = 163840
EROWS = EP // 128
D0, D1, D2 = 256, 256, 128
NC, NS = 2, 16
RT = NP // NS
K = 128
CH1 = EP // NS // K
CH2 = EP // (NC * NS) // K

_MESH = plsc.VectorSubcoreMesh(core_axis_name="c", subcore_axis_name="s",
                               num_cores=NC, num_subcores=NS)


def _deg_body(dst2d, zeros_k, ones_k, p0, p1, stage_d, ones_v, acc):
  cid = lax.axis_index("c")
  sid = lax.axis_index("s")
  r0 = sid * RT

  pltpu.sync_copy(zeros_k, ones_v)

  def init_chunk(j, carry):
    pltpu.sync_copy(ones_v, acc.at[pl.ds(r0 + j * K, K)])
    return carry

  lax.fori_loop(0, RT // K, init_chunk, 0)
  pltpu.sync_copy(ones_k, ones_v)
  pltpu.sync_copy(dst2d.at[pl.ds((cid * NS + sid) * CH2, CH2)], stage_d)
  plsc.subcore_barrier()

  def chunk(j, carry):
    pltpu.sync_copy(ones_v, acc.at[stage_d.at[j]], add=True)
    return carry

  lax.fori_loop(0, CH2, chunk, 0)
  plsc.subcore_barrier()

  def out_chunk(j, carry):
    sl = pl.ds(r0 + j * K, K)
    pltpu.sync_copy(acc.at[sl], ones_v)

    @pl.when(cid == 0)
    def _():
      pltpu.sync_copy(ones_v, p0.at[sl])

    @pl.when(cid == 1)
    def _():
      pltpu.sync_copy(ones_v, p1.at[sl])

    return carry

  lax.fori_loop(0, RT // K, out_chunk, 0)


def _deg_counts(dst2d):
  zeros_k = jnp.zeros((K, 128), jnp.float32)
  ones_k = jnp.ones((K, 128), jnp.float32)
  f = pl.kernel(
      _deg_body,
      out_type=(jax.ShapeDtypeStruct((NP, 128), jnp.float32),
                jax.ShapeDtypeStruct((NP, 128), jnp.float32)),
      mesh=_MESH,
      scratch_types=[
          pltpu.VMEM((CH2, 128), jnp.int32),
          pltpu.VMEM((K, 128), jnp.float32),
          pltpu.VMEM_SHARED((NP, 128), jnp.float32),
      ],
  )
  return f(dst2d, zeros_k, ones_k)


def _msg1_body(ya, yb, srcp, dstp, oa, ob, sv0, sv1, dv0, dv1, buf0, buf1,
               acc, sem0, sem1):
  cid = lax.axis_index("c")
  sid = lax.axis_index("s")
  r0 = sid * RT
  svs = (sv0, sv1)
  dvs = (dv0, dv1)
  bufs = (buf0, buf1)
  sems = (sem0, sem1)

  def init_chunk(j, carry):
    sl = pl.ds(r0 + j * K, K)

    @pl.when(cid == 0)
    def _():
      pltpu.sync_copy(ya.at[sl], buf0)

    @pl.when(cid == 1)
    def _():
      pltpu.sync_copy(yb.at[sl], buf0)

    pltpu.sync_copy(buf0, acc.at[sl])
    return carry

  lax.fori_loop(0, RT // K, init_chunk, 0)
  plsc.subcore_barrier()

  def gather(tab, b):
    pltpu.async_copy(tab.at[svs[b].at[pl.ds(0, K // 2)]],
                     bufs[b].at[pl.ds(0, K // 2)], sems[b])
    pltpu.async_copy(tab.at[svs[b].at[pl.ds(K // 2, K // 2)]],
                     bufs[b].at[pl.ds(K // 2, K // 2)], sems[b])

  def gather_wait(b):
    pltpu.make_async_copy(ya.at[svs[b].at[pl.ds(0, K // 2)]],
                          bufs[b].at[pl.ds(0, K // 2)], sems[b]).wait()
    pltpu.make_async_copy(ya.at[svs[b].at[pl.ds(K // 2, K // 2)]],
                          bufs[b].at[pl.ds(K // 2, K // 2)], sems[b]).wait()

  e0 = sid * (EP // NS)
  for b in range(2):
    pltpu.sync_copy(srcp.at[pl.ds(e0 + b * K, K)], svs[b])
    pltpu.sync_copy(dstp.at[pl.ds(e0 + b * K, K)], dvs[b])

    @pl.when(cid == 0)
    def _():
      gather(ya, b)

    @pl.when(cid == 1)
    def _():
      gather(yb, b)

  def chunk(i, carry):
    j = i * 2
    for b in range(2):
      jb = j + b
      gather_wait(b)
      pltpu.sync_copy(bufs[b], acc.at[dvs[b]], add=True)

      @pl.when(jb + 2 < CH1)
      def _():
        pltpu.sync_copy(srcp.at[pl.ds(e0 + (jb + 2) * K, K)], svs[b])
        pltpu.sync_copy(dstp.at[pl.ds(e0 + (jb + 2) * K, K)], dvs[b])

        @pl.when(cid == 0)
        def _():
          gather(ya, b)

        @pl.when(cid == 1)
        def _():
          gather(yb, b)

    return carry

  lax.fori_loop(0, CH1 // 2, chunk, 0)
  plsc.subcore_barrier()

  def out_chunk(j, carry):
    sl = pl.ds(r0 + j * K, K)
    pltpu.sync_copy(acc.at[sl], buf0)

    @pl.when(cid == 0)
    def _():
      pltpu.sync_copy(buf0, oa.at[sl])

    @pl.when(cid == 1)
    def _():
      pltpu.sync_copy(buf0, ob.at[sl])

    return carry

  lax.fori_loop(0, RT // K, out_chunk, 0)


def _message_pass1(ya, yb, srcp, dstp):
  f = pl.kernel(
      _msg1_body,
      out_type=(jax.ShapeDtypeStruct((NP, 128), jnp.float32),
                jax.ShapeDtypeStruct((NP, 128), jnp.float32)),
      mesh=_MESH,
      scratch_types=[
          pltpu.VMEM((K,), jnp.int32),
          pltpu.VMEM((K,), jnp.int32),
          pltpu.VMEM((K,), jnp.int32),
          pltpu.VMEM((K,), jnp.int32),
          pltpu.VMEM((K, 128), jnp.float32),
          pltpu.VMEM((K, 128), jnp.float32),
          pltpu.VMEM_SHARED((NP, 128), jnp.float32),
          pltpu.SemaphoreType.DMA,
          pltpu.SemaphoreType.DMA,
      ],
  )
  return f(ya, yb, srcp, dstp)


def _msg2_body(y2, srcp, dstp, zeros_k, p0, p1, sv0, sv1, dv0, dv1, buf0,
               buf1, acc, sem0, sem1):
  cid = lax.axis_index("c")
  sid = lax.axis_index("s")
  r0 = sid * RT
  svs = (sv0, sv1)
  dvs = (dv0, dv1)
  bufs = (buf0, buf1)
  sems = (sem0, sem1)

  @pl.when(cid == 1)
  def _():
    pltpu.sync_copy(zeros_k, buf0)

  def init_chunk(j, carry):
    sl = pl.ds(r0 + j * K, K)

    @pl.when(cid == 0)
    def _():
      pltpu.sync_copy(y2.at[sl], buf0)

    pltpu.sync_copy(buf0, acc.at[sl])
    return carry

  lax.fori_loop(0, RT // K, init_chunk, 0)
  plsc.subcore_barrier()

  e0 = (cid * NS + sid) * (EP // (NC * NS))
  for b in range(2):
    pltpu.sync_copy(srcp.at[pl.ds(e0 + b * K, K)], svs[b])
    pltpu.sync_copy(dstp.at[pl.ds(e0 + b * K, K)], dvs[b])
    pltpu.async_copy(y2.at[svs[b]], bufs[b], sems[b])

  def chunk(i, carry):
    j = i * 2
    for b in range(2):
      jb = j + b
      pltpu.make_async_copy(y2.at[svs[b]], bufs[b], sems[b]).wait()
      pltpu.sync_copy(bufs[b], acc.at[dvs[b]], add=True)

      @pl.when(jb + 2 < CH2)
      def _():
        pltpu.sync_copy(srcp.at[pl.ds(e0 + (jb + 2) * K, K)], svs[b])
        pltpu.sync_copy(dstp.at[pl.ds(e0 + (jb + 2) * K, K)], dvs[b])
        pltpu.async_copy(y2.at[svs[b]], bufs[b], sems[b])

    return carry

  lax.fori_loop(0, CH2 // 2, chunk, 0)
  plsc.subcore_barrier()

  def out_chunk(j, carry):
    sl = pl.ds(r0 + j * K, K)
    pltpu.sync_copy(acc.at[sl], buf0)

    @pl.when(cid == 0)
    def _():
      pltpu.sync_copy(buf0, p0.at[sl])

    @pl.when(cid == 1)
    def _():
      pltpu.sync_copy(buf0, p1.at[sl])

    return carry

  lax.fori_loop(0, RT // K, out_chunk, 0)


def _message_pass2(y2, srcp, dstp):
  zeros_k = jnp.zeros((K, 128), jnp.float32)
  f = pl.kernel(
      _msg2_body,
      out_type=(jax.ShapeDtypeStruct((NP, 128), jnp.float32),
                jax.ShapeDtypeStruct((NP, 128), jnp.float32)),
      mesh=_MESH,
      scratch_types=[
          pltpu.VMEM((K,), jnp.int32),
          pltpu.VMEM((K,), jnp.int32),
          pltpu.VMEM((K,), jnp.int32),
          pltpu.VMEM((K,), jnp.int32),
          pltpu.VMEM((K, 128), jnp.float32),
          pltpu.VMEM((K, 128), jnp.float32),
          pltpu.VMEM_SHARED((NP, 128), jnp.float32),
          pltpu.SemaphoreType.DMA,
          pltpu.SemaphoreType.DMA,
      ],
  )
  return f(y2, srcp, dstp, zeros_k)


_RB = 1000


def _mm1_body(d0, d1, x_ref, w_ref, ya, yb):
  dinv = lax.rsqrt(d0[...] + d1[...] + 1.0)
  y = jnp.dot(x_ref[...], w_ref[...],
              preferred_element_type=jnp.float32) * dinv
  ya[...] = y[:, :D1 // 2]
  yb[...] = y[:, D1 // 2:]


def _layer1_matmul(deg0, deg1, x, W1):
  grid = (N // _RB,)
  return pl.pallas_call(
      _mm1_body,
      grid=grid,
      in_specs=[
          pl.BlockSpec((_RB, 1), lambda i: (i, 0)),
          pl.BlockSpec((_RB, 1), lambda i: (i, 0)),
          pl.BlockSpec((_RB, D0), lambda i: (i, 0)),
          pl.BlockSpec((D0, D1), lambda i: (0, 0)),
      ],
      out_specs=(
          pl.BlockSpec((_RB, D1 // 2), lambda i: (i, 0)),
          pl.BlockSpec((_RB, D1 // 2), lambda i: (i, 0)),
      ),
      out_shape=(
          jax.ShapeDtypeStruct((NP, D1 // 2), jnp.float32),
          jax.ShapeDtypeStruct((NP, D1 // 2), jnp.float32),
      ),
  )(deg0, deg1, x, W1)


def _mm2_body(aa, ab, d0, d1, b1, w_ref, y2):
  dinv = lax.rsqrt(d0[...] + d1[...] + 1.0)
  h = jnp.concatenate([aa[...], ab[...]], axis=1)
  h = jnp.maximum(h * dinv + b1[...], 0.0)
  y2[...] = jnp.dot(h, w_ref[...], preferred_element_type=jnp.float32) * dinv


def _layer2_matmul(acc1a, acc1b, deg0, deg1, b1, W2):
  grid = (N // _RB,)
  return pl.pallas_call(
      _mm2_body,
      grid=grid,
      in_specs=[
          pl.BlockSpec((_RB, D1 // 2), lambda i: (i, 0)),
          pl.BlockSpec((_RB, D1 // 2), lambda i: (i, 0)),
          pl.BlockSpec((_RB, 1), lambda i: (i, 0)),
          pl.BlockSpec((_RB, 1), lambda i: (i, 0)),
          pl.BlockSpec((1, D1), lambda i: (0, 0)),
          pl.BlockSpec((D1, D2), lambda i: (0, 0)),
      ],
      out_specs=pl.BlockSpec((_RB, D2), lambda i: (i, 0)),
      out_shape=jax.ShapeDtypeStruct((NP, D2), jnp.float32),
  )(acc1a, acc1b, deg0, deg1, b1, W2)


def _epi_body(p0, p1, d0, d1, b2, z_ref):
  dinv = lax.rsqrt(d0[...] + d1[...] + 1.0)
  z_ref[...] = (p0[...] + p1[...]) * dinv + b2[...]


def _epilogue(p0, p1, deg0, deg1, b2):
  grid = (N // _RB,)
  return pl.pallas_call(
      _epi_body,
      grid=grid,
      in_specs=[
          pl.BlockSpec((_RB, D2), lambda i: (i, 0)),
          pl.BlockSpec((_RB, D2), lambda i: (i, 0)),
          pl.BlockSpec((_RB, 1), lambda i: (i, 0)),
          pl.BlockSpec((_RB, 1), lambda i: (i, 0)),
          pl.BlockSpec((1, D2), lambda i: (0, 0)),
      ],
      out_specs=pl.BlockSpec((_RB, D2), lambda i: (i, 0)),
      out_shape=jax.ShapeDtypeStruct((N, D2), jnp.float32),
  )(p0, p1, deg0, deg1, b2)


@jax.jit
def kernel(x, edge_index, W1, b1, W2, b2):
  pad = jnp.full((EP - E,), N, jnp.int32)
  srcp = jnp.concatenate([edge_index[0].astype(jnp.int32), pad])
  dstp = jnp.concatenate([edge_index[1].astype(jnp.int32), pad])
  dst2d = dstp.reshape(EROWS, 128)

  h0, h1 = _deg_counts(dst2d)
  deg0 = h0[:, :1]
  deg1 = h1[:, :1]
  y1a, y1b = _layer1_matmul(deg0, deg1, x, W1)
  acc1a, acc1b = _message_pass1(y1a, y1b, srcp, dstp)
  y2 = _layer2_matmul(acc1a, acc1b, deg0, deg1, b1.reshape(1, D1), W2)
  p0, p1 = _message_pass2(y2, srcp, dstp)
  return _epilogue(p0, p1, deg0, deg1, b2.reshape(1, D2))


# speedup vs baseline: 1.1007x; 1.0145x over previous
import functools
import jax
import jax.numpy as jnp
import numpy as np
from jax import lax
from jax.experimental import pallas as pl
from jax.experimental.pallas import tpu as pltpu
from jax.experimental.pallas import tpu_sc as plsc

N = 10000
NP = 10240
E = 160000
EP = 163840
EROWS = EP // 128
D0, D1, D2 = 256, 256, 128
NC, NS = 2, 16
RT = NP // NS
K = 128
CH1 = EP // NS // K
CH2 = EP // (NC * NS) // K

_MESH = plsc.VectorSubcoreMesh(core_axis_name="c", subcore_axis_name="s",
                               num_cores=NC, num_subcores=NS)


def _deg_body(dst2d, zeros_k, ones_k, p0, p1, stage_d, ones_v, acc):
  cid = lax.axis_index("c")
  sid = lax.axis_index("s")
  r0 = sid * RT

  pltpu.sync_copy(zeros_k, ones_v)

  def init_chunk(j, carry):
    pltpu.sync_copy(ones_v, acc.at[pl.ds(r0 + j * K, K)])
    return carry

  lax.fori_loop(0, RT // K, init_chunk, 0)
  pltpu.sync_copy(ones_k, ones_v)
  pltpu.sync_copy(dst2d.at[pl.ds((cid * NS + sid) * CH2, CH2)], stage_d)
  plsc.subcore_barrier()

  def chunk(j, carry):
    pltpu.sync_copy(ones_v, acc.at[stage_d.at[j]], add=True)
    return carry

  lax.fori_loop(0, CH2, chunk, 0)
  plsc.subcore_barrier()

  def out_chunk(j, carry):
    sl = pl.ds(r0 + j * K, K)
    pltpu.sync_copy(acc.at[sl], ones_v)

    @pl.when(cid == 0)
    def _():
      pltpu.sync_copy(ones_v, p0.at[sl])

    @pl.when(cid == 1)
    def _():
      pltpu.sync_copy(ones_v, p1.at[sl])

    return carry

  lax.fori_loop(0, RT // K, out_chunk, 0)


def _deg_counts(dst2d):
  zeros_k = jnp.zeros((K, 128), jnp.float32)
  ones_k = jnp.ones((K, 128), jnp.float32)
  f = pl.kernel(
      _deg_body,
      out_type=(jax.ShapeDtypeStruct((NP, 128), jnp.float32),
                jax.ShapeDtypeStruct((NP, 128), jnp.float32)),
      mesh=_MESH,
      scratch_types=[
          pltpu.VMEM((CH2, 128), jnp.int32),
          pltpu.VMEM((K, 128), jnp.float32),
          pltpu.VMEM_SHARED((NP, 128), jnp.float32),
      ],
  )
  return f(dst2d, zeros_k, ones_k)


def _msg1_body(ya, yb, srcp, dstp, oa, ob, sv0, sv1, dv0, dv1, buf0, buf1,
               acc, sem0, sem1):
  cid = lax.axis_index("c")
  sid = lax.axis_index("s")
  r0 = sid * RT
  svs = (sv0, sv1)
  dvs = (dv0, dv1)
  bufs = (buf0, buf1)
  sems = (sem0, sem1)

  def init_chunk(j, carry):
    sl = pl.ds(r0 + j * K, K)

    @pl.when(cid == 0)
    def _():
      pltpu.sync_copy(ya.at[sl], buf0)

    @pl.when(cid == 1)
    def _():
      pltpu.sync_copy(yb.at[sl], buf0)

    pltpu.sync_copy(buf0, acc.at[sl])
    return carry

  lax.fori_loop(0, RT // K, init_chunk, 0)
  plsc.subcore_barrier()

  def gather(tab, b):
    pltpu.async_copy(tab.at[svs[b].at[pl.ds(0, K // 2)]],
                     bufs[b].at[pl.ds(0, K // 2)], sems[b])
    pltpu.async_copy(tab.at[svs[b].at[pl.ds(K // 2, K // 2)]],
                     bufs[b].at[pl.ds(K // 2, K // 2)], sems[b])

  def gather_wait(b):
    pltpu.make_async_copy(ya.at[svs[b].at[pl.ds(0, K // 2)]],
                          bufs[b].at[pl.ds(0, K // 2)], sems[b]).wait()
    pltpu.make_async_copy(ya.at[svs[b].at[pl.ds(K // 2, K // 2)]],
                          bufs[b].at[pl.ds(K // 2, K // 2)], sems[b]).wait()

  e0 = sid * (EP // NS)
  for b in range(2):
    pltpu.sync_copy(srcp.at[pl.ds(e0 + b * K, K)], svs[b])
    pltpu.sync_copy(dstp.at[pl.ds(e0 + b * K, K)], dvs[b])

    @pl.when(cid == 0)
    def _():
      gather(ya, b)

    @pl.when(cid == 1)
    def _():
      gather(yb, b)

  def chunk(i, carry):
    j = i * 2
    for b in range(2):
      jb = j + b
      gather_wait(b)
      pltpu.sync_copy(bufs[b], acc.at[dvs[b]], add=True)

      @pl.when(jb + 2 < CH1)
      def _():
        pltpu.sync_copy(srcp.at[pl.ds(e0 + (jb + 2) * K, K)], svs[b])
        pltpu.sync_copy(dstp.at[pl.ds(e0 + (jb + 2) * K, K)], dvs[b])

        @pl.when(cid == 0)
        def _():
          gather(ya, b)

        @pl.when(cid == 1)
        def _():
          gather(yb, b)

    return carry

  lax.fori_loop(0, CH1 // 2, chunk, 0)
  plsc.subcore_barrier()

  def out_chunk(j, carry):
    sl = pl.ds(r0 + j * K, K)
    pltpu.sync_copy(acc.at[sl], buf0)

    @pl.when(cid == 0)
    def _():
      pltpu.sync_copy(buf0, oa.at[sl])

    @pl.when(cid == 1)
    def _():
      pltpu.sync_copy(buf0, ob.at[sl])

    return carry

  lax.fori_loop(0, RT // K, out_chunk, 0)


def _message_pass1(ya, yb, srcp, dstp):
  f = pl.kernel(
      _msg1_body,
      out_type=(jax.ShapeDtypeStruct((NP, 128), jnp.float32),
                jax.ShapeDtypeStruct((NP, 128), jnp.float32)),
      mesh=_MESH,
      scratch_types=[
          pltpu.VMEM((K,), jnp.int32),
          pltpu.VMEM((K,), jnp.int32),
          pltpu.VMEM((K,), jnp.int32),
          pltpu.VMEM((K,), jnp.int32),
          pltpu.VMEM((K, 128), jnp.float32),
          pltpu.VMEM((K, 128), jnp.float32),
          pltpu.VMEM_SHARED((NP, 128), jnp.float32),
          pltpu.SemaphoreType.DMA,
          pltpu.SemaphoreType.DMA,
      ],
  )
  return f(ya, yb, srcp, dstp)


def _msg2_body(y2, y2c, srcp, dstp, zeros_k, p0, p1, sv0, sv1, dv0, dv1,
               buf0, buf1, acc, sem0, sem1):
  cid = lax.axis_index("c")
  sid = lax.axis_index("s")
  r0 = sid * RT
  svs = (sv0, sv1)
  dvs = (dv0, dv1)
  bufs = (buf0, buf1)
  sems = (sem0, sem1)

  @pl.when(cid == 1)
  def _():
    pltpu.sync_copy(zeros_k, buf0)

  def init_chunk(j, carry):
    sl = pl.ds(r0 + j * K, K)

    @pl.when(cid == 0)
    def _():
      pltpu.sync_copy(y2.at[sl], buf0)

    pltpu.sync_copy(buf0, acc.at[sl])
    return carry

  lax.fori_loop(0, RT // K, init_chunk, 0)
  plsc.subcore_barrier()

  def gather(b):
    @pl.when(cid == 0)
    def _():
      pltpu.async_copy(y2.at[svs[b]], bufs[b], sems[b])

    @pl.when(cid == 1)
    def _():
      pltpu.async_copy(y2c.at[svs[b]], bufs[b], sems[b])

  e0 = (cid * NS + sid) * (EP // (NC * NS))
  for b in range(2):
    pltpu.sync_copy(srcp.at[pl.ds(e0 + b * K, K)], svs[b])
    pltpu.sync_copy(dstp.at[pl.ds(e0 + b * K, K)], dvs[b])
    gather(b)

  def chunk(i, carry):
    j = i * 2
    for b in range(2):
      jb = j + b
      pltpu.make_async_copy(y2.at[svs[b]], bufs[b], sems[b]).wait()
      pltpu.sync_copy(bufs[b], acc.at[dvs[b]], add=True)

      @pl.when(jb + 2 < CH2)
      def _():
        pltpu.sync_copy(srcp.at[pl.ds(e0 + (jb + 2) * K, K)], svs[b])
        pltpu.sync_copy(dstp.at[pl.ds(e0 + (jb + 2) * K, K)], dvs[b])
        gather(b)

    return carry

  lax.fori_loop(0, CH2 // 2, chunk, 0)
  plsc.subcore_barrier()

  def out_chunk(j, carry):
    sl = pl.ds(r0 + j * K, K)
    pltpu.sync_copy(acc.at[sl], buf0)

    @pl.when(cid == 0)
    def _():
      pltpu.sync_copy(buf0, p0.at[sl])

    @pl.when(cid == 1)
    def _():
      pltpu.sync_copy(buf0, p1.at[sl])

    return carry

  lax.fori_loop(0, RT // K, out_chunk, 0)


def _message_pass2(y2, y2c, srcp, dstp):
  zeros_k = jnp.zeros((K, 128), jnp.float32)
  f = pl.kernel(
      _msg2_body,
      out_type=(jax.ShapeDtypeStruct((NP, 128), jnp.float32),
                jax.ShapeDtypeStruct((NP, 128), jnp.float32)),
      mesh=_MESH,
      scratch_types=[
          pltpu.VMEM((K,), jnp.int32),
          pltpu.VMEM((K,), jnp.int32),
          pltpu.VMEM((K,), jnp.int32),
          pltpu.VMEM((K,), jnp.int32),
          pltpu.VMEM((K, 128), jnp.float32),
          pltpu.VMEM((K, 128), jnp.float32),
          pltpu.VMEM_SHARED((NP, 128), jnp.float32),
          pltpu.SemaphoreType.DMA,
          pltpu.SemaphoreType.DMA,
      ],
  )
  return f(y2, y2c, srcp, dstp, zeros_k)


_RB = 1000


def _mm1_body(d0, d1, x_ref, w_ref, ya, yb):
  dinv = lax.rsqrt(d0[...] + d1[...] + 1.0)
  y = jnp.dot(x_ref[...], w_ref[...],
              preferred_element_type=jnp.float32) * dinv
  ya[...] = y[:, :D1 // 2]
  yb[...] = y[:, D1 // 2:]


def _layer1_matmul(deg0, deg1, x, W1):
  grid = (N // _RB,)
  return pl.pallas_call(
      _mm1_body,
      grid=grid,
      in_specs=[
          pl.BlockSpec((_RB, 1), lambda i: (i, 0)),
          pl.BlockSpec((_RB, 1), lambda i: (i, 0)),
          pl.BlockSpec((_RB, D0), lambda i: (i, 0)),
          pl.BlockSpec((D0, D1), lambda i: (0, 0)),
      ],
      out_specs=(
          pl.BlockSpec((_RB, D1 // 2), lambda i: (i, 0)),
          pl.BlockSpec((_RB, D1 // 2), lambda i: (i, 0)),
      ),
      out_shape=(
          jax.ShapeDtypeStruct((NP, D1 // 2), jnp.float32),
          jax.ShapeDtypeStruct((NP, D1 // 2), jnp.float32),
      ),
  )(deg0, deg1, x, W1)


def _mm2_body(aa, ab, d0, d1, b1, w_ref, y2, y2c):
  dinv = lax.rsqrt(d0[...] + d1[...] + 1.0)
  h = jnp.concatenate([aa[...], ab[...]], axis=1)
  h = jnp.maximum(h * dinv + b1[...], 0.0)
  y = jnp.dot(h, w_ref[...], preferred_element_type=jnp.float32) * dinv
  y2[...] = y
  y2c[...] = y


def _layer2_matmul(acc1a, acc1b, deg0, deg1, b1, W2):
  grid = (N // _RB,)
  return pl.pallas_call(
      _mm2_body,
      grid=grid,
      in_specs=[
          pl.BlockSpec((_RB, D1 // 2), lambda i: (i, 0)),
          pl.BlockSpec((_RB, D1 // 2), lambda i: (i, 0)),
          pl.BlockSpec((_RB, 1), lambda i: (i, 0)),
          pl.BlockSpec((_RB, 1), lambda i: (i, 0)),
          pl.BlockSpec((1, D1), lambda i: (0, 0)),
          pl.BlockSpec((D1, D2), lambda i: (0, 0)),
      ],
      out_specs=(
          pl.BlockSpec((_RB, D2), lambda i: (i, 0)),
          pl.BlockSpec((_RB, D2), lambda i: (i, 0)),
      ),
      out_shape=(
          jax.ShapeDtypeStruct((NP, D2), jnp.float32),
          jax.ShapeDtypeStruct((NP, D2), jnp.float32),
      ),
  )(acc1a, acc1b, deg0, deg1, b1, W2)


def _epi_body(p0, p1, d0, d1, b2, z_ref):
  dinv = lax.rsqrt(d0[...] + d1[...] + 1.0)
  z_ref[...] = (p0[...] + p1[...]) * dinv + b2[...]


def _epilogue(p0, p1, deg0, deg1, b2):
  grid = (N // _RB,)
  return pl.pallas_call(
      _epi_body,
      grid=grid,
      in_specs=[
          pl.BlockSpec((_RB, D2), lambda i: (i, 0)),
          pl.BlockSpec((_RB, D2), lambda i: (i, 0)),
          pl.BlockSpec((_RB, 1), lambda i: (i, 0)),
          pl.BlockSpec((_RB, 1), lambda i: (i, 0)),
          pl.BlockSpec((1, D2), lambda i: (0, 0)),
      ],
      out_specs=pl.BlockSpec((_RB, D2), lambda i: (i, 0)),
      out_shape=jax.ShapeDtypeStruct((N, D2), jnp.float32),
  )(p0, p1, deg0, deg1, b2)


@jax.jit
def kernel(x, edge_index, W1, b1, W2, b2):
  pad = jnp.full((EP - E,), N, jnp.int32)
  srcp = jnp.concatenate([edge_index[0].astype(jnp.int32), pad])
  dstp = jnp.concatenate([edge_index[1].astype(jnp.int32), pad])
  dst2d = dstp.reshape(EROWS, 128)

  h0, h1 = _deg_counts(dst2d)
  deg0 = h0[:, :1]
  deg1 = h1[:, :1]
  y1a, y1b = _layer1_matmul(deg0, deg1, x, W1)
  acc1a, acc1b = _message_pass1(y1a, y1b, srcp, dstp)
  y2, y2c = _layer2_matmul(acc1a, acc1b, deg0, deg1, b1.reshape(1, D1), W2)
  p0, p1 = _message_pass2(y2, y2c, srcp, dstp)
  return _epilogue(p0, p1, deg0, deg1, b2.reshape(1, D2))
